# baseline (speedup 1.0000x reference)
import functools

import jax
import jax.numpy as jnp
from jax import lax
from jax.experimental import pallas as pl
from jax.experimental.pallas import tpu as pltpu
from jax.experimental.pallas import tpu_sc as plsc

N_NODES = 100000
N_EDGES = 800000
AD = 16
BD = 8

NC, NS = 2, 16
NW = NC * NS
EPW = N_EDGES // NW
RPC = 15
CHUNK = RPC * 128
N_FULL = EPW // CHUNK
TAIL = EPW - N_FULL * CHUNK

NROWS = N_EDGES // 128
HALF = N_NODES // NC
RPS = NROWS // NS
EXTRA = NROWS - RPS * NS
NPT = HALF // NS

@functools.cache
def _make_sc_gather():
    mesh = plsc.VectorSubcoreMesh(core_axis_name="c", subcore_axis_name="s")
    return pl.kernel(
        _sc_gather_body,
        out_type=jax.ShapeDtypeStruct((N_EDGES, AD), jnp.float32),
        mesh=mesh,
        scratch_types=[
            pltpu.VMEM((CHUNK,), jnp.int32),
            pltpu.VMEM((CHUNK, AD), jnp.float32),
            pltpu.SemaphoreType.DMA,
        ],
        compiler_params=pltpu.CompilerParams(use_tc_tiling_on_sc=False),
    )


def _sc_gather_body(atom_hbm, src_hbm, out_hbm, idx_v, rows_v, sem):
    wid = lax.axis_index("s") * NC + lax.axis_index("c")
    base = wid * EPW

    def chunk_body(c, carry):
        off = base + c * CHUNK
        pltpu.sync_copy(src_hbm.at[pl.ds(off, CHUNK)], idx_v)
        copies = [
            pltpu.async_copy(
                atom_hbm.at[idx_v.at[pl.ds(j * 128, 128)]],
                rows_v.at[pl.ds(j * 128, 128)],
                sem,
            )
            for j in range(RPC)
        ]
        for d in copies:
            d.wait()
        pltpu.sync_copy(rows_v, out_hbm.at[pl.ds(off, CHUNK)])
        return carry

    lax.fori_loop(0, N_FULL, chunk_body, 0)

    off = base + N_FULL * CHUNK
    pltpu.sync_copy(src_hbm.at[pl.ds(off, TAIL)], idx_v.at[pl.ds(0, TAIL)])
    pltpu.async_copy(
        atom_hbm.at[idx_v.at[pl.ds(0, TAIL)]], rows_v.at[pl.ds(0, TAIL)], sem
    ).wait()
    pltpu.sync_copy(rows_v.at[pl.ds(0, TAIL)], out_hbm.at[pl.ds(off, TAIL)])


@functools.cache
def _make_sc_scatter():
    mesh = plsc.VectorSubcoreMesh(core_axis_name="c", subcore_axis_name="s")
    return pl.kernel(
        _sc_scatter_body,
        out_type=jax.ShapeDtypeStruct((N_NODES, AD), jnp.float32),
        mesh=mesh,
        scratch_types=[
            pltpu.VMEM((RPC, 128), jnp.int32),
            pltpu.VMEM((RPC, 128, AD), jnp.float32),
            pltpu.VMEM_SHARED((HALF + 1, AD), jnp.float32),
            pltpu.SemaphoreType.DMA,
        ],
        compiler_params=pltpu.CompilerParams(use_tc_tiling_on_sc=False),
    )


def _sc_scatter_body(t_hbm, idx_hbm, zeros_hbm, out_hbm, idx_v, rows_v, acc, sem):
    cid = lax.axis_index("c")
    sid = lax.axis_index("s")

    pltpu.sync_copy(zeros_hbm.at[pl.ds(0, NPT)], acc.at[pl.ds(sid * NPT, NPT)])

    @pl.when(sid == 0)
    def _zero_dump():
        pltpu.sync_copy(zeros_hbm.at[pl.ds(NPT, 1)], acc.at[pl.ds(HALF, 1)])

    plsc.subcore_barrier()

    row0 = RPS * sid + jnp.minimum(sid, EXTRA)

    def chunk_body(c, carry):
        r0 = row0 + c * RPC
        pltpu.sync_copy(idx_hbm.at[cid, pl.ds(r0, RPC)], idx_v)
        pltpu.sync_copy(t_hbm.at[pl.ds(r0, RPC)], rows_v)
        copies = [
            pltpu.async_copy(rows_v.at[j], acc.at[idx_v.at[j]], sem, add=True)
            for j in range(RPC)
        ]
        for d in copies:
            d.wait()
        return carry

    lax.fori_loop(0, RPS // RPC, chunk_body, 0)

    @pl.when(sid < EXTRA)
    def _tail():
        r0 = row0 + RPS
        pltpu.sync_copy(idx_hbm.at[cid, pl.ds(r0, 1)], idx_v.at[pl.ds(0, 1)])
        pltpu.sync_copy(t_hbm.at[pl.ds(r0, 1)], rows_v.at[pl.ds(0, 1)])
        pltpu.sync_copy(rows_v.at[0], acc.at[idx_v.at[0]], add=True)

    plsc.subcore_barrier()
    pltpu.sync_copy(
        acc.at[pl.ds(sid * NPT, NPT)],
        out_hbm.at[pl.ds(cid * HALF + sid * NPT, NPT)],
    )


T_EDGE = 4000


def _tc_bilinear_body(bond_ref, a_ref, r_ref, w_ref, b_ref, out_ref):
    bond = bond_ref[...]
    a = a_ref[...]
    u = jnp.dot(bond, r_ref[...], preferred_element_type=jnp.float32)
    u = u * jnp.tile(a, (1, BD))
    out = jnp.dot(u, w_ref[...], preferred_element_type=jnp.float32)
    out = out + jnp.dot(a, b_ref[...], preferred_element_type=jnp.float32)
    out_ref[...] = out


def _tc_bilinear(bond, gathered, r_mat, w_mat, b_mat):
    return pl.pallas_call(
        _tc_bilinear_body,
        grid=(N_EDGES // T_EDGE,),
        in_specs=[
            pl.BlockSpec((T_EDGE, BD), lambda i: (i, 0)),
            pl.BlockSpec((T_EDGE, AD), lambda i: (i, 0)),
            pl.BlockSpec((BD, BD * AD), lambda i: (0, 0)),
            pl.BlockSpec((BD * AD, AD), lambda i: (0, 0)),
            pl.BlockSpec((AD, AD), lambda i: (0, 0)),
        ],
        out_specs=pl.BlockSpec((T_EDGE, AD), lambda i: (i, 0)),
        out_shape=jax.ShapeDtypeStruct((N_EDGES, AD), jnp.float32),
        compiler_params=pltpu.CompilerParams(
            dimension_semantics=("arbitrary",),
        ),
    )(bond, gathered, r_mat, w_mat, b_mat)


def _tc_idx_body(dst_ref, out_ref):
    lo = pl.program_id(0) * HALF
    v = dst_ref[...]
    in_range = (v >= lo) & (v < lo + HALF)
    out_ref[0] = jnp.where(in_range, v - lo, HALF)


def _tc_localize(dst2d):
    return pl.pallas_call(
        _tc_idx_body,
        grid=(NC,),
        in_specs=[pl.BlockSpec((NROWS, 128), lambda i: (0, 0))],
        out_specs=pl.BlockSpec((1, NROWS, 128), lambda i: (i, 0, 0)),
        out_shape=jax.ShapeDtypeStruct((NC, NROWS, 128), jnp.int32),
        compiler_params=pltpu.CompilerParams(
            dimension_semantics=("arbitrary",),
        ),
    )(dst2d)


def kernel(atom_features, bond_features, pair_indices, kernel, bias):
    pi = pair_indices.astype(jnp.int32)
    dst = pi[:, 0]
    src = pi[:, 1]

    w_mat = kernel.reshape(BD, AD, AD).transpose(0, 2, 1).reshape(BD * AD, AD)
    b_mat = bias.reshape(AD, AD).T
    r_mat = jnp.kron(jnp.eye(BD, dtype=jnp.float32), jnp.ones((1, AD), jnp.float32))

    gathered = _make_sc_gather()(atom_features, src)
    transformed = _tc_bilinear(bond_features, gathered, r_mat, w_mat, b_mat)

    t3d = transformed.reshape(NROWS, 128, AD)
    dst2d = dst.reshape(NROWS, 128)
    idx2 = _tc_localize(dst2d)
    zeros = jnp.zeros((NPT + 1, AD), jnp.float32)
    return _make_sc_scatter()(t3d, idx2, zeros)


# speedup vs baseline: 1.4250x; 1.4250x over previous
import functools

import jax
import jax.numpy as jnp
from jax import lax
from jax.experimental import pallas as pl
from jax.experimental.pallas import tpu as pltpu
from jax.experimental.pallas import tpu_sc as plsc

N_NODES = 100000
N_EDGES = 800000
AD = 16
BD = 8

NC, NS = 2, 16
NW = NC * NS
EPW = N_EDGES // NW
RPC = 15
CHUNK = RPC * 128
N_FULL = EPW // CHUNK
TAIL = EPW - N_FULL * CHUNK

NROWS = N_EDGES // 128
HALF = N_NODES // NC
RPS = NROWS // NS
EXTRA = NROWS - RPS * NS
NPT = HALF // NS

@functools.cache
def _make_sc_gather():
    mesh = plsc.VectorSubcoreMesh(core_axis_name="c", subcore_axis_name="s")
    return pl.kernel(
        _sc_gather_body,
        out_type=jax.ShapeDtypeStruct((N_EDGES, AD), jnp.float32),
        mesh=mesh,
        scratch_types=[
            pltpu.VMEM((CHUNK,), jnp.int32),
            pltpu.VMEM((CHUNK, AD), jnp.float32),
            pltpu.SemaphoreType.DMA,
        ],
        compiler_params=pltpu.CompilerParams(use_tc_tiling_on_sc=False),
    )


def _sc_gather_body(atom_hbm, src_hbm, out_hbm, idx_v, rows_v, sem):
    wid = lax.axis_index("s") * NC + lax.axis_index("c")
    base = wid * EPW

    def chunk_body(c, carry):
        off = base + c * CHUNK
        pltpu.sync_copy(src_hbm.at[pl.ds(off, CHUNK)], idx_v)
        copies = [
            pltpu.async_copy(
                atom_hbm.at[idx_v.at[pl.ds(j * 128, 128)]],
                rows_v.at[pl.ds(j * 128, 128)],
                sem,
            )
            for j in range(RPC)
        ]
        for d in copies:
            d.wait()
        pltpu.sync_copy(rows_v, out_hbm.at[pl.ds(off, CHUNK)])
        return carry

    lax.fori_loop(0, N_FULL, chunk_body, 0)

    off = base + N_FULL * CHUNK
    pltpu.sync_copy(src_hbm.at[pl.ds(off, TAIL)], idx_v.at[pl.ds(0, TAIL)])
    pltpu.async_copy(
        atom_hbm.at[idx_v.at[pl.ds(0, TAIL)]], rows_v.at[pl.ds(0, TAIL)], sem
    ).wait()
    pltpu.sync_copy(rows_v.at[pl.ds(0, TAIL)], out_hbm.at[pl.ds(off, TAIL)])


@functools.cache
def _make_sc_scatter():
    mesh = plsc.VectorSubcoreMesh(core_axis_name="c", subcore_axis_name="s")
    return pl.kernel(
        _sc_scatter_body,
        out_type=jax.ShapeDtypeStruct((N_NODES, AD), jnp.float32),
        mesh=mesh,
        scratch_types=[
            pltpu.VMEM((RPC, 128), jnp.int32),
            pltpu.VMEM((RPC, 128, AD), jnp.float32),
            pltpu.VMEM_SHARED((HALF + 1, AD), jnp.float32),
            pltpu.SemaphoreType.DMA,
        ],
        compiler_params=pltpu.CompilerParams(use_tc_tiling_on_sc=False),
    )


def _sc_scatter_body(t_hbm, idx_hbm, zeros_hbm, out_hbm, idx_v, rows_v, acc, sem):
    cid = lax.axis_index("c")
    sid = lax.axis_index("s")

    pltpu.sync_copy(zeros_hbm.at[pl.ds(0, NPT)], acc.at[pl.ds(sid * NPT, NPT)])

    @pl.when(sid == 0)
    def _zero_dump():
        pltpu.sync_copy(zeros_hbm.at[pl.ds(NPT, 1)], acc.at[pl.ds(HALF, 1)])

    plsc.subcore_barrier()

    row0 = RPS * sid + jnp.minimum(sid, EXTRA)

    def chunk_body(c, carry):
        r0 = row0 + c * RPC
        pltpu.sync_copy(idx_hbm.at[cid, pl.ds(r0, RPC)], idx_v)
        pltpu.sync_copy(t_hbm.at[pl.ds(r0, RPC)], rows_v)
        copies = [
            pltpu.async_copy(rows_v.at[j], acc.at[idx_v.at[j]], sem, add=True)
            for j in range(RPC)
        ]
        for d in copies:
            d.wait()
        return carry

    lax.fori_loop(0, RPS // RPC, chunk_body, 0)

    @pl.when(sid < EXTRA)
    def _tail():
        r0 = row0 + RPS
        pltpu.sync_copy(idx_hbm.at[cid, pl.ds(r0, 1)], idx_v.at[pl.ds(0, 1)])
        pltpu.sync_copy(t_hbm.at[pl.ds(r0, 1)], rows_v.at[pl.ds(0, 1)])
        pltpu.sync_copy(rows_v.at[0], acc.at[idx_v.at[0]], add=True)

    plsc.subcore_barrier()
    pltpu.sync_copy(
        acc.at[pl.ds(sid * NPT, NPT)],
        out_hbm.at[pl.ds(cid * HALF + sid * NPT, NPT)],
    )


T_EDGE = 6400


def _tc_bilinear_body(bt_ref, at_ref, r_ref, t_ref, w_ref, b_ref, out_ref):
    dn = (((0,), (0,)), ((), ()))
    f32 = jnp.float32
    expand = lax.dot_general(r_ref[...], bt_ref[...], dn, preferred_element_type=f32)
    tile = lax.dot_general(t_ref[...], at_ref[...], dn, preferred_element_type=f32)
    u = expand * tile
    out = lax.dot_general(w_ref[...], u, dn, preferred_element_type=f32)
    out = out + lax.dot_general(b_ref[...], at_ref[...], dn, preferred_element_type=f32)
    out_ref[...] = out


def _tc_bilinear(bond_t, a_t, r_mat, t_mat, w_mat, b_mat):
    return pl.pallas_call(
        _tc_bilinear_body,
        grid=(N_EDGES // T_EDGE,),
        in_specs=[
            pl.BlockSpec((BD, T_EDGE), lambda i: (0, i)),
            pl.BlockSpec((AD, T_EDGE), lambda i: (0, i)),
            pl.BlockSpec((BD, BD * AD), lambda i: (0, 0)),
            pl.BlockSpec((AD, BD * AD), lambda i: (0, 0)),
            pl.BlockSpec((BD * AD, AD), lambda i: (0, 0)),
            pl.BlockSpec((AD, AD), lambda i: (0, 0)),
        ],
        out_specs=pl.BlockSpec((AD, T_EDGE), lambda i: (0, i)),
        out_shape=jax.ShapeDtypeStruct((AD, N_EDGES), jnp.float32),
        compiler_params=pltpu.CompilerParams(
            dimension_semantics=("arbitrary",),
        ),
    )(bond_t, a_t, r_mat, t_mat, w_mat, b_mat)


def _tc_idx_body(dst_ref, out_ref):
    lo = pl.program_id(0) * HALF
    v = dst_ref[...]
    in_range = (v >= lo) & (v < lo + HALF)
    out_ref[0] = jnp.where(in_range, v - lo, HALF)


def _tc_localize(dst2d):
    return pl.pallas_call(
        _tc_idx_body,
        grid=(NC,),
        in_specs=[pl.BlockSpec((NROWS, 128), lambda i: (0, 0))],
        out_specs=pl.BlockSpec((1, NROWS, 128), lambda i: (i, 0, 0)),
        out_shape=jax.ShapeDtypeStruct((NC, NROWS, 128), jnp.int32),
        compiler_params=pltpu.CompilerParams(
            dimension_semantics=("arbitrary",),
        ),
    )(dst2d)


def kernel(atom_features, bond_features, pair_indices, kernel, bias):
    pi = pair_indices.astype(jnp.int32)
    dst = pi[:, 0]
    src = pi[:, 1]

    w_mat = kernel.reshape(BD, AD, AD).transpose(0, 2, 1).reshape(BD * AD, AD)
    b_mat = bias.reshape(AD, AD).T
    r_mat = jnp.kron(jnp.eye(BD, dtype=jnp.float32), jnp.ones((1, AD), jnp.float32))
    t_mat = jnp.kron(jnp.ones((1, BD), jnp.float32), jnp.eye(AD, dtype=jnp.float32))

    gathered = _make_sc_gather()(atom_features, src)
    out_t = _tc_bilinear(bond_features.T, gathered.T, r_mat, t_mat, w_mat, b_mat)

    t3d = out_t.T.reshape(NROWS, 128, AD)
    dst2d = dst.reshape(NROWS, 128)
    idx2 = _tc_localize(dst2d)
    zeros = jnp.zeros((NPT + 1, AD), jnp.float32)
    return _make_sc_scatter()(t3d, idx2, zeros)


# speedup vs baseline: 1.8771x; 1.3173x over previous
import functools

import jax
import jax.numpy as jnp
from jax import lax
from jax.experimental import pallas as pl
from jax.experimental.pallas import tpu as pltpu
from jax.experimental.pallas import tpu_sc as plsc

N_NODES = 100000
N_EDGES = 800000
AD = 16
BD = 8

NC, NS = 2, 16
NW = NC * NS
EPW = N_EDGES // NW
RPC = 15
CHUNK = RPC * 128
N_FULL = EPW // CHUNK
TAIL = EPW - N_FULL * CHUNK

NROWS = N_EDGES // 128
HALF = N_NODES // NC
RPS = NROWS // NS
EXTRA = NROWS - RPS * NS
NPT = HALF // NS
NDUMP = 128

@functools.cache
def _make_sc_gather():
    mesh = plsc.VectorSubcoreMesh(core_axis_name="c", subcore_axis_name="s")
    return pl.kernel(
        _sc_gather_body,
        out_type=jax.ShapeDtypeStruct((N_EDGES, AD), jnp.float32),
        mesh=mesh,
        scratch_types=[
            pltpu.VMEM((CHUNK,), jnp.int32),
            pltpu.VMEM((CHUNK, AD), jnp.float32),
            pltpu.SemaphoreType.DMA,
        ],
        compiler_params=pltpu.CompilerParams(use_tc_tiling_on_sc=False),
    )


def _sc_gather_body(atom_hbm, src_hbm, out_hbm, idx_v, rows_v, sem):
    wid = lax.axis_index("s") * NC + lax.axis_index("c")
    base = wid * EPW

    def chunk_body(c, carry):
        off = base + c * CHUNK
        pltpu.sync_copy(src_hbm.at[pl.ds(off, CHUNK)], idx_v)
        copies = [
            pltpu.async_copy(
                atom_hbm.at[idx_v.at[pl.ds(j * 128, 128)]],
                rows_v.at[pl.ds(j * 128, 128)],
                sem,
            )
            for j in range(RPC)
        ]
        for d in copies:
            d.wait()
        pltpu.sync_copy(rows_v, out_hbm.at[pl.ds(off, CHUNK)])
        return carry

    lax.fori_loop(0, N_FULL, chunk_body, 0)

    off = base + N_FULL * CHUNK
    pltpu.sync_copy(src_hbm.at[pl.ds(off, TAIL)], idx_v.at[pl.ds(0, TAIL)])
    pltpu.async_copy(
        atom_hbm.at[idx_v.at[pl.ds(0, TAIL)]], rows_v.at[pl.ds(0, TAIL)], sem
    ).wait()
    pltpu.sync_copy(rows_v.at[pl.ds(0, TAIL)], out_hbm.at[pl.ds(off, TAIL)])


@functools.cache
def _make_sc_scatter():
    mesh = plsc.VectorSubcoreMesh(core_axis_name="c", subcore_axis_name="s")
    return pl.kernel(
        _sc_scatter_body,
        out_type=jax.ShapeDtypeStruct((N_NODES, AD), jnp.float32),
        mesh=mesh,
        scratch_types=[
            pltpu.VMEM((RPC, 128), jnp.int32),
            pltpu.VMEM((RPC, 128, AD), jnp.float32),
            pltpu.VMEM_SHARED((HALF + NDUMP, AD), jnp.float32),
            pltpu.SemaphoreType.DMA,
        ],
        compiler_params=pltpu.CompilerParams(use_tc_tiling_on_sc=False),
    )


def _sc_scatter_body(t_hbm, idx_hbm, zeros_hbm, out_hbm, idx_v, rows_v, acc, sem):
    cid = lax.axis_index("c")
    sid = lax.axis_index("s")

    pltpu.sync_copy(zeros_hbm.at[pl.ds(0, NPT)], acc.at[pl.ds(sid * NPT, NPT)])

    @pl.when(sid == 0)
    def _zero_dump():
        pltpu.sync_copy(zeros_hbm.at[pl.ds(0, NDUMP)], acc.at[pl.ds(HALF, NDUMP)])

    plsc.subcore_barrier()

    row0 = RPS * sid + jnp.minimum(sid, EXTRA)

    def chunk_body(c, carry):
        r0 = row0 + c * RPC
        pltpu.sync_copy(idx_hbm.at[cid, pl.ds(r0, RPC)], idx_v)
        pltpu.sync_copy(t_hbm.at[pl.ds(r0, RPC)], rows_v)
        copies = [
            pltpu.async_copy(rows_v.at[j], acc.at[idx_v.at[j]], sem, add=True)
            for j in range(RPC)
        ]
        for d in copies:
            d.wait()
        return carry

    lax.fori_loop(0, RPS // RPC, chunk_body, 0)

    @pl.when(sid < EXTRA)
    def _tail():
        r0 = row0 + RPS
        pltpu.sync_copy(idx_hbm.at[cid, pl.ds(r0, 1)], idx_v.at[pl.ds(0, 1)])
        pltpu.sync_copy(t_hbm.at[pl.ds(r0, 1)], rows_v.at[pl.ds(0, 1)])
        pltpu.sync_copy(rows_v.at[0], acc.at[idx_v.at[0]], add=True)

    plsc.subcore_barrier()
    pltpu.sync_copy(
        acc.at[pl.ds(sid * NPT, NPT)],
        out_hbm.at[pl.ds(cid * HALF + sid * NPT, NPT)],
    )


T_EDGE = 6400


def _tc_bilinear_body(bt_ref, at_ref, r_ref, t_ref, w_ref, b_ref, out_ref):
    dn = (((0,), (0,)), ((), ()))
    f32 = jnp.float32
    expand = lax.dot_general(r_ref[...], bt_ref[...], dn, preferred_element_type=f32)
    tile = lax.dot_general(t_ref[...], at_ref[...], dn, preferred_element_type=f32)
    u = expand * tile
    out = lax.dot_general(w_ref[...], u, dn, preferred_element_type=f32)
    out = out + lax.dot_general(b_ref[...], at_ref[...], dn, preferred_element_type=f32)
    out_ref[...] = out


def _tc_bilinear(bond_t, a_t, r_mat, t_mat, w_mat, b_mat):
    return pl.pallas_call(
        _tc_bilinear_body,
        grid=(N_EDGES // T_EDGE,),
        in_specs=[
            pl.BlockSpec((BD, T_EDGE), lambda i: (0, i)),
            pl.BlockSpec((AD, T_EDGE), lambda i: (0, i)),
            pl.BlockSpec((BD, BD * AD), lambda i: (0, 0)),
            pl.BlockSpec((AD, BD * AD), lambda i: (0, 0)),
            pl.BlockSpec((BD * AD, AD), lambda i: (0, 0)),
            pl.BlockSpec((AD, AD), lambda i: (0, 0)),
        ],
        out_specs=pl.BlockSpec((AD, T_EDGE), lambda i: (0, i)),
        out_shape=jax.ShapeDtypeStruct((AD, N_EDGES), jnp.float32),
        compiler_params=pltpu.CompilerParams(
            dimension_semantics=("arbitrary",),
        ),
    )(bond_t, a_t, r_mat, t_mat, w_mat, b_mat)


def _tc_idx_body(dst_ref, out_ref):
    lo = pl.program_id(0) * HALF
    v = dst_ref[...]
    in_range = (v >= lo) & (v < lo + HALF)
    lane = lax.broadcasted_iota(jnp.int32, (NROWS, 128), 1)
    out_ref[0] = jnp.where(in_range, v - lo, HALF + lane)


def _tc_localize(dst2d):
    return pl.pallas_call(
        _tc_idx_body,
        grid=(NC,),
        in_specs=[pl.BlockSpec((NROWS, 128), lambda i: (0, 0))],
        out_specs=pl.BlockSpec((1, NROWS, 128), lambda i: (i, 0, 0)),
        out_shape=jax.ShapeDtypeStruct((NC, NROWS, 128), jnp.int32),
        compiler_params=pltpu.CompilerParams(
            dimension_semantics=("arbitrary",),
        ),
    )(dst2d)


def kernel(atom_features, bond_features, pair_indices, kernel, bias):
    pi = pair_indices.astype(jnp.int32)
    dst = pi[:, 0]
    src = pi[:, 1]

    w_mat = kernel.reshape(BD, AD, AD).transpose(0, 2, 1).reshape(BD * AD, AD)
    b_mat = bias.reshape(AD, AD).T
    r_mat = jnp.kron(jnp.eye(BD, dtype=jnp.float32), jnp.ones((1, AD), jnp.float32))
    t_mat = jnp.kron(jnp.ones((1, BD), jnp.float32), jnp.eye(AD, dtype=jnp.float32))

    gathered = _make_sc_gather()(atom_features, src)
    out_t = _tc_bilinear(bond_features.T, gathered.T, r_mat, t_mat, w_mat, b_mat)

    t3d = out_t.T.reshape(NROWS, 128, AD)
    dst2d = dst.reshape(NROWS, 128)
    idx2 = _tc_localize(dst2d)
    zeros = jnp.zeros((NPT, AD), jnp.float32)
    return _make_sc_scatter()(t3d, idx2, zeros)
